# Initial kernel scaffold; baseline (speedup 1.0000x reference)
import jax
import jax.numpy as jnp
from jax.experimental import pallas as pl


def kernel(inputs, table):
    raise NotImplementedError("write your pallas kernel here")


# speedup vs baseline: 3.7591x; 3.7591x over previous
import functools

import jax
import jax.numpy as jnp
from jax import lax
from jax.experimental import pallas as pl
from jax.experimental.pallas import tpu as pltpu
from jax.experimental.pallas import tpu_sc as plsc

L = 16
W = 128


def _masked_gather(table, idx):
    V, D = table.shape
    N = idx.shape[1]
    mesh = plsc.VectorSubcoreMesh(core_axis_name="c", subcore_axis_name="s")

    @functools.partial(
        pl.kernel,
        out_type=jax.ShapeDtypeStruct((N, D), table.dtype),
        mesh=mesh,
        compiler_params=pltpu.CompilerParams(
            needs_layout_passes=False, use_tc_tiling_on_sc=False
        ),
    )
    def gather_kernel(table_hbm, idx_hbm, out_hbm):
        def body(i_vmem, o_vmem):
            pltpu.sync_copy(table_hbm.at[i_vmem.at[0]], o_vmem)

            minv = i_vmem[0, pl.ds(0, L)]
            for g in range(1, W // L):
                minv = jnp.minimum(minv, i_vmem[0, pl.ds(g * L, L)])

            @pl.when(jnp.min(minv) == 0)
            def _fixup():
                @pl.loop(0, W // L)
                def _(g):
                    iv = i_vmem[0, pl.ds(g * L, L)]

                    @pl.when(jnp.min(iv) == 0)
                    def _group():
                        zero = jnp.zeros((L,), jnp.int32)
                        for r in range(L):
                            row = g * L + r
                            iv_r = plsc.load_gather(i_vmem, [zero, zero + row])
                            is_zero = iv_r == 0
                            for c in range(D // L):
                                sl = (row, pl.ds(c * L, L))
                                o_vmem[sl] = jnp.where(is_zero, 0.0, o_vmem[sl])

        pltpu.emit_pipeline(
            body,
            grid=(N // W,),
            in_specs=[pl.BlockSpec((1, W), lambda i: (0, i))],
            out_specs=[pl.BlockSpec((W, D), lambda i: (i, 0))],
            core_axis_name=("c", "s"),
            dimension_semantics=(pltpu.PARALLEL,),
        )(idx_hbm, out_hbm)

    return gather_kernel(table, idx)


def kernel(inputs, table):
    B, H = inputs.shape
    D = table.shape[1]
    idx = inputs.reshape(1, B * H).astype(jnp.int32)
    out = _masked_gather(table, idx)
    return out.reshape(B, H, D)


# speedup vs baseline: 4.2671x; 1.1351x over previous
import functools

import jax
import jax.numpy as jnp
from jax import lax
from jax.experimental import pallas as pl
from jax.experimental.pallas import tpu as pltpu
from jax.experimental.pallas import tpu_sc as plsc

L = 16
G = 128
GPW = 4
W = G * GPW


def _masked_gather(table, idx):
    V, D = table.shape
    N = idx.shape[0] * idx.shape[1]
    mesh = plsc.VectorSubcoreMesh(core_axis_name="c", subcore_axis_name="s")

    @functools.partial(
        pl.kernel,
        out_type=jax.ShapeDtypeStruct((N, D), table.dtype),
        mesh=mesh,
        scratch_types=[pltpu.SemaphoreType.DMA],
        compiler_params=pltpu.CompilerParams(
            needs_layout_passes=False, use_tc_tiling_on_sc=False
        ),
    )
    def gather_kernel(table_hbm, idx_hbm, out_hbm, gsem):
        def body(i_vmem, o_vmem):
            for j in range(GPW):
                pltpu.async_copy(
                    table_hbm.at[i_vmem.at[j]],
                    o_vmem.at[pl.ds(j * G, G)],
                    gsem,
                )
            for j in range(GPW):
                pltpu.make_async_copy(
                    table_hbm.at[i_vmem.at[j]],
                    o_vmem.at[pl.ds(j * G, G)],
                    gsem,
                ).wait()

            minv = i_vmem[0, pl.ds(0, L)]
            for j in range(GPW):
                for g in range(1 if j == 0 else 0, G // L):
                    minv = jnp.minimum(minv, i_vmem[j, pl.ds(g * L, L)])

            @pl.when(jnp.min(minv) == 0)
            def _fixup():
                @pl.loop(0, GPW)
                def _(j):
                    @pl.loop(0, G // L)
                    def _(g):
                        iv = i_vmem[j, pl.ds(g * L, L)]

                        @pl.when(jnp.min(iv) == 0)
                        def _group():
                            zero = jnp.zeros((L,), jnp.int32)
                            for r in range(L):
                                iv_r = plsc.load_gather(
                                    i_vmem, [zero + j, zero + (g * L + r)]
                                )
                                is_zero = iv_r == 0
                                row = j * G + g * L + r
                                for c in range(D // L):
                                    sl = (row, pl.ds(c * L, L))
                                    o_vmem[sl] = jnp.where(
                                        is_zero, 0.0, o_vmem[sl]
                                    )

        pltpu.emit_pipeline(
            body,
            grid=(N // W,),
            in_specs=[pl.BlockSpec((GPW, G), lambda i: (i, 0))],
            out_specs=[pl.BlockSpec((W, D), lambda i: (i, 0))],
            core_axis_name=("c", "s"),
            dimension_semantics=(pltpu.PARALLEL,),
        )(idx_hbm, out_hbm)

    return gather_kernel(table, idx)


def kernel(inputs, table):
    B, H = inputs.shape
    D = table.shape[1]
    idx = inputs.reshape(B * H // G, G).astype(jnp.int32)
    out = _masked_gather(table, idx)
    return out.reshape(B, H, D)


# speedup vs baseline: 4.2795x; 1.0029x over previous
import functools

import jax
import jax.numpy as jnp
from jax import lax
from jax.experimental import pallas as pl
from jax.experimental.pallas import tpu as pltpu
from jax.experimental.pallas import tpu_sc as plsc

L = 16
RB = 4


def _splits(hist):
    out, off = [], 0
    while off < hist:
        n = min(128, hist - off)
        out.append((off, n))
        off += n
    return out


def _group_offsets(hist):
    offs = list(range(0, hist - L + 1, L))
    if offs[-1] + L < hist:
        offs.append(hist - L)
    return offs


def kernel(inputs, table):
    B, H = inputs.shape
    V, D = table.shape
    idx = inputs.astype(jnp.int32)
    mesh = plsc.VectorSubcoreMesh(core_axis_name="c", subcore_axis_name="s")
    chunks = _splits(H)
    goffs = _group_offsets(H)

    @functools.partial(
        pl.kernel,
        out_type=jax.ShapeDtypeStruct((B, H, D), table.dtype),
        mesh=mesh,
        scratch_types=[pltpu.SemaphoreType.DMA],
        compiler_params=pltpu.CompilerParams(
            needs_layout_passes=False, use_tc_tiling_on_sc=False
        ),
    )
    def gather_kernel(table_hbm, idx_hbm, out_hbm, gsem):
        def body(i_vmem, o_vmem):
            for r in range(RB):
                for off, n in chunks:
                    pltpu.async_copy(
                        table_hbm.at[i_vmem.at[r, pl.ds(off, n)]],
                        o_vmem.at[r, pl.ds(off, n)],
                        gsem,
                    )
            for r in range(RB):
                for off, n in chunks:
                    pltpu.make_async_copy(
                        table_hbm.at[i_vmem.at[r, pl.ds(off, n)]],
                        o_vmem.at[r, pl.ds(off, n)],
                        gsem,
                    ).wait()

            minv = i_vmem[0, pl.ds(0, L)]
            for r in range(RB):
                for g in goffs[1 if r == 0 else 0:]:
                    minv = jnp.minimum(minv, i_vmem[r, pl.ds(g, L)])

            ngroups = len(goffs)
            last_off = goffs[-1]

            @pl.when(jnp.min(minv) == 0)
            def _fixup():
                @pl.loop(0, RB)
                def _(r):
                    @pl.loop(0, ngroups)
                    def _(gi):
                        g = jnp.minimum(gi * L, last_off)
                        iv = i_vmem[r, pl.ds(g, L)]

                        @pl.when(jnp.min(iv) == 0)
                        def _group():
                            zero = jnp.zeros((L,), jnp.int32)
                            for l in range(L):
                                iv_l = plsc.load_gather(
                                    i_vmem, [zero + r, zero + (g + l)]
                                )
                                is_zero = iv_l == 0
                                for c in range(D // L):
                                    sl = (r, g + l, pl.ds(c * L, L))
                                    o_vmem[sl] = jnp.where(
                                        is_zero, 0.0, o_vmem[sl]
                                    )

        pltpu.emit_pipeline(
            body,
            grid=(B // RB,),
            in_specs=[pl.BlockSpec((RB, H), lambda i: (i, 0))],
            out_specs=[pl.BlockSpec((RB, H, D), lambda i: (i, 0, 0))],
            core_axis_name=("c", "s"),
            dimension_semantics=(pltpu.PARALLEL,),
        )(idx_hbm, out_hbm)

    return gather_kernel(table, idx)


# speedup vs baseline: 5.5814x; 1.3042x over previous
import functools

import jax
import jax.numpy as jnp
from jax import lax
from jax.experimental import pallas as pl
from jax.experimental.pallas import tpu as pltpu
from jax.experimental.pallas import tpu_sc as plsc

L = 16
NBUF = 3
NC, NS = 2, 16
NW = NC * NS


def _splits(hist):
    out, off = [], 0
    while off < hist:
        n = min(128, hist - off)
        out.append((off, n))
        off += n
    return out


def _group_offsets(hist):
    offs = list(range(0, hist - L + 1, L))
    if offs[-1] + L < hist:
        offs.append(hist - L)
    return offs


def kernel(inputs, table):
    B, H = inputs.shape
    V, D = table.shape
    BPT = B // NW
    HP = (H + L - 1) // L * L
    idx = inputs.astype(jnp.int32)
    if HP != H:
        idx = jnp.pad(idx, ((0, 0), (0, HP - H)), constant_values=1)
    idx = idx.reshape(B * HP)
    table_w = jnp.pad(table, ((0, 0), (0, 128 - D)))
    mesh = plsc.VectorSubcoreMesh(core_axis_name="c", subcore_axis_name="s")
    chunks = _splits(H)
    goffs = list(range(0, HP, L))
    ngroups = len(goffs)
    last_off = goffs[-1]

    @functools.partial(
        pl.kernel,
        out_type=jax.ShapeDtypeStruct((B, H, 128), table.dtype),
        mesh=mesh,
        scratch_types=[
            pltpu.VMEM((BPT * HP,), jnp.int32),
            pltpu.VMEM((HP, 128), jnp.float32),
            pltpu.VMEM((HP, 128), jnp.float32),
            pltpu.VMEM((HP, 128), jnp.float32),
            pltpu.SemaphoreType.DMA,
            pltpu.SemaphoreType.DMA,
            pltpu.SemaphoreType.DMA,
            pltpu.SemaphoreType.DMA,
            pltpu.SemaphoreType.DMA,
            pltpu.SemaphoreType.DMA,
            pltpu.SemaphoreType.DMA,
        ],
        compiler_params=pltpu.CompilerParams(
            needs_layout_passes=False, use_tc_tiling_on_sc=True
        ),
    )
    def gather_kernel(table_hbm, idx_hbm, out_hbm, idx_v, w0, w1, w2,
                      isem, g0, g1, g2, s0, s1, s2):
        wid = lax.axis_index("s") * NC + lax.axis_index("c")
        base = wid * BPT
        bufs, gsems, wsems = [w0, w1, w2], [g0, g1, g2], [s0, s1, s2]

        pltpu.async_copy(
            idx_hbm.at[pl.ds(base * HP, BPT * HP)], idx_v, isem
        ).wait()

        def issue_gathers(b, row):
            for off, n in chunks:
                pltpu.async_copy(
                    table_hbm.at[idx_v.at[pl.ds(row * HP + off, n)]],
                    bufs[b].at[pl.ds(off, n)],
                    gsems[b],
                )

        def wait_gathers(b):
            for off, n in chunks:
                pltpu.make_async_copy(
                    table_hbm.at[idx_v.at[pl.ds(off, n)]],
                    bufs[b].at[pl.ds(off, n)],
                    gsems[b],
                ).wait()

        def issue_write(b, row):
            pltpu.async_copy(
                bufs[b].at[pl.ds(0, H)], out_hbm.at[base + row], wsems[b]
            )

        def wait_write(b):
            pltpu.make_async_copy(
                bufs[b].at[pl.ds(0, H)], out_hbm.at[base], wsems[b]
            ).wait()

        def fixup(b, row):
            rbase = row * HP
            minv = idx_v[pl.ds(rbase, L)]
            for g in goffs[1:]:
                minv = jnp.minimum(minv, idx_v[pl.ds(rbase + g, L)])

            @pl.when(jnp.min(minv) == 0)
            def _fix():
                @pl.loop(0, ngroups)
                def _(gi):
                    g = jnp.minimum(gi * L, last_off)
                    iv = idx_v[pl.ds(rbase + g, L)]

                    @pl.when(jnp.min(iv) == 0)
                    def _group():
                        lane = lax.iota(jnp.int32, L)
                        for l in range(L):
                            zl = jnp.min(jnp.where(lane == l, iv, 1))

                            @pl.when(zl == 0)
                            def _zero_row(l=l):
                                for c in range(D // L):
                                    sl = (g + l, pl.ds(c * L, L))
                                    bufs[b][sl] = jnp.zeros(
                                        (L,), jnp.float32
                                    )

        issue_gathers(0, 0)
        issue_gathers(1, 1)

        main = BPT - BPT % NBUF

        @pl.loop(0, main, step=NBUF)
        def _(r0):
            for j in range(NBUF):
                row = r0 + j
                b = j
                wait_gathers(b)
                fixup(b, row)
                issue_write(b, row)
                nxt = row + 2
                bn = (j + 2) % NBUF

                @pl.when(nxt < BPT)
                def _(row=row, nxt=nxt, bn=bn):
                    @pl.when(row >= 1)
                    def _():
                        wait_write(bn)

                    issue_gathers(bn, nxt)

        for row in range(main, BPT):
            b = row % NBUF
            wait_gathers(b)
            fixup(b, row)
            issue_write(b, row)

        for b in range(NBUF):
            wait_write(b)

    return gather_kernel(table_w, idx)[:, :, :D]


# speedup vs baseline: 5.5897x; 1.0015x over previous
import functools

import jax
import jax.numpy as jnp
from jax import lax
from jax.experimental import pallas as pl
from jax.experimental.pallas import tpu as pltpu
from jax.experimental.pallas import tpu_sc as plsc

L = 16
NBUF = 4
NC, NS = 2, 16
NW = NC * NS


def _splits(hist):
    out, off = [], 0
    while off < hist:
        n = min(128, hist - off)
        out.append((off, n))
        off += n
    return out


def _group_offsets(hist):
    offs = list(range(0, hist - L + 1, L))
    if offs[-1] + L < hist:
        offs.append(hist - L)
    return offs


def kernel(inputs, table):
    B, H = inputs.shape
    V, D = table.shape
    BPT = B // NW
    HP = (H + L - 1) // L * L
    idx = inputs.astype(jnp.int32)
    if HP != H:
        idx = jnp.pad(idx, ((0, 0), (0, HP - H)), constant_values=1)
    idx = idx.reshape(B * HP)
    table_w = jnp.pad(table, ((0, 0), (0, 128 - D)))
    mesh = plsc.VectorSubcoreMesh(core_axis_name="c", subcore_axis_name="s")
    chunks = _splits(H)
    goffs = list(range(0, HP, L))
    ngroups = len(goffs)
    last_off = goffs[-1]

    @functools.partial(
        pl.kernel,
        out_type=jax.ShapeDtypeStruct((B, H, 128), table.dtype),
        mesh=mesh,
        scratch_types=[
            pltpu.VMEM((BPT * HP,), jnp.int32),
            pltpu.VMEM((H, 128), jnp.float32),
            pltpu.VMEM((H, 128), jnp.float32),
            pltpu.VMEM((H, 128), jnp.float32),
            pltpu.VMEM((H, 128), jnp.float32),
            pltpu.SemaphoreType.DMA,
            pltpu.SemaphoreType.DMA,
            pltpu.SemaphoreType.DMA,
            pltpu.SemaphoreType.DMA,
            pltpu.SemaphoreType.DMA,
            pltpu.SemaphoreType.DMA,
            pltpu.SemaphoreType.DMA,
            pltpu.SemaphoreType.DMA,
            pltpu.SemaphoreType.DMA,
        ],
        compiler_params=pltpu.CompilerParams(
            needs_layout_passes=False, use_tc_tiling_on_sc=True
        ),
    )
    def gather_kernel(table_hbm, idx_hbm, out_hbm, idx_v, w0, w1, w2, w3,
                      isem, g0, g1, g2, g3, s0, s1, s2, s3):
        wid = lax.axis_index("s") * NC + lax.axis_index("c")
        base = wid * BPT
        bufs = [w0, w1, w2, w3]
        gsems, wsems = [g0, g1, g2, g3], [s0, s1, s2, s3]

        pltpu.async_copy(
            idx_hbm.at[pl.ds(base * HP, BPT * HP)], idx_v, isem
        ).wait()

        def issue_gathers(b, row):
            for off, n in chunks:
                pltpu.async_copy(
                    table_hbm.at[idx_v.at[pl.ds(row * HP + off, n)]],
                    bufs[b].at[pl.ds(off, n)],
                    gsems[b],
                )

        def wait_gathers(b):
            for off, n in chunks:
                pltpu.make_async_copy(
                    table_hbm.at[idx_v.at[pl.ds(off, n)]],
                    bufs[b].at[pl.ds(off, n)],
                    gsems[b],
                ).wait()

        def issue_write(b, row):
            pltpu.async_copy(bufs[b], out_hbm.at[base + row], wsems[b])

        def wait_write(b):
            pltpu.make_async_copy(
                bufs[b], out_hbm.at[base], wsems[b]
            ).wait()

        def fixup(b, row):
            rbase = row * HP
            minv = idx_v[pl.ds(rbase, L)]
            for g in goffs[1:]:
                minv = jnp.minimum(minv, idx_v[pl.ds(rbase + g, L)])

            @pl.when(jnp.min(minv) == 0)
            def _fix():
                @pl.loop(0, ngroups)
                def _(gi):
                    g = jnp.minimum(gi * L, last_off)
                    iv = idx_v[pl.ds(rbase + g, L)]

                    @pl.when(jnp.min(iv) == 0)
                    def _group():
                        lane = lax.iota(jnp.int32, L)
                        for l in range(L):
                            zl = jnp.min(jnp.where(lane == l, iv, 1))

                            @pl.when(zl == 0)
                            def _zero_row(l=l):
                                for c in range(D // L):
                                    sl = (g + l, pl.ds(c * L, L))
                                    bufs[b][sl] = jnp.zeros(
                                        (L,), jnp.float32
                                    )

        issue_gathers(0, 0)
        issue_gathers(1, 1)
        issue_gathers(2, 2)

        main = BPT - BPT % NBUF

        @pl.loop(0, main, step=NBUF)
        def _(r0):
            for j in range(NBUF):
                row = r0 + j
                b = j
                wait_gathers(b)
                fixup(b, row)
                issue_write(b, row)
                nxt = row + 3
                bn = (j + 3) % NBUF

                @pl.when(nxt < BPT)
                def _(row=row, nxt=nxt, bn=bn):
                    @pl.when(row >= 1)
                    def _():
                        wait_write(bn)

                    issue_gathers(bn, nxt)

        for row in range(main, BPT):
            b = row % NBUF
            wait_gathers(b)
            fixup(b, row)
            issue_write(b, row)

        for b in range(NBUF):
            wait_write(b)

    return gather_kernel(table_w, idx)[:, :, :D]


# speedup vs baseline: 5.5934x; 1.0007x over previous
import functools

import jax
import jax.numpy as jnp
from jax import lax
from jax.experimental import pallas as pl
from jax.experimental.pallas import tpu as pltpu
from jax.experimental.pallas import tpu_sc as plsc

L = 16
NBUF = 4
NC, NS = 2, 16
NW = NC * NS


def _splits(hist):
    out, off = [], 0
    while off < hist:
        n = min(128, hist - off)
        out.append((off, n))
        off += n
    return out


def kernel(inputs, table):
    B, H = inputs.shape
    V, D = table.shape
    BPT = B // NW
    HP = (H + L - 1) // L * L
    idx = inputs.astype(jnp.int32)
    if HP != H:
        idx = jnp.pad(idx, ((0, 0), (0, HP - H)), constant_values=1)
    idx = idx.reshape(B * HP)
    table_w = jnp.pad(table, ((0, 0), (0, 128 - D)))
    mesh = plsc.VectorSubcoreMesh(core_axis_name="c", subcore_axis_name="s")
    chunks = _splits(H)
    goffs = list(range(0, HP, L))
    ngroups = len(goffs)
    last_off = goffs[-1]

    @functools.partial(
        pl.kernel,
        out_type=jax.ShapeDtypeStruct((B, H, 128), table.dtype),
        mesh=mesh,
        scratch_types=[
            pltpu.VMEM((BPT * HP,), jnp.int32),
            pltpu.VMEM((H, 128), jnp.float32),
            pltpu.VMEM((H, 128), jnp.float32),
            pltpu.VMEM((H, 128), jnp.float32),
            pltpu.VMEM((H, 128), jnp.float32),
            pltpu.SemaphoreType.DMA,
            pltpu.SemaphoreType.DMA,
            pltpu.SemaphoreType.DMA,
            pltpu.SemaphoreType.DMA,
            pltpu.SemaphoreType.DMA,
            pltpu.SemaphoreType.DMA,
            pltpu.SemaphoreType.DMA,
            pltpu.SemaphoreType.DMA,
            pltpu.SemaphoreType.DMA,
        ],
        compiler_params=pltpu.CompilerParams(
            needs_layout_passes=False, use_tc_tiling_on_sc=True
        ),
    )
    def gather_kernel(table_hbm, idx_hbm, out_hbm, idx_v, w0, w1, w2, w3,
                      isem, g0, g1, g2, g3, s0, s1, s2, s3):
        wid = lax.axis_index("s") * NC + lax.axis_index("c")
        base = wid * BPT
        bufs = [w0, w1, w2, w3]
        gsems, wsems = [g0, g1, g2, g3], [s0, s1, s2, s3]

        pltpu.async_copy(
            idx_hbm.at[pl.ds(base * HP, BPT * HP)], idx_v, isem
        ).wait()

        def issue_gathers(b, row):
            for off, n in chunks:
                pltpu.async_copy(
                    table_hbm.at[idx_v.at[pl.ds(row * HP + off, n)]],
                    bufs[b].at[pl.ds(off, n)],
                    gsems[b],
                )

        def wait_gathers(b):
            for off, n in chunks:
                pltpu.make_async_copy(
                    table_hbm.at[idx_v.at[pl.ds(off, n)]],
                    bufs[b].at[pl.ds(off, n)],
                    gsems[b],
                ).wait()

        def issue_write(b, row):
            pltpu.async_copy(bufs[b], out_hbm.at[base + row], wsems[b])

        def wait_write(b):
            pltpu.make_async_copy(
                bufs[b], out_hbm.at[base], wsems[b]
            ).wait()

        def fixup(b, row):
            rbase = row * HP
            minv = idx_v[pl.ds(rbase, L)]
            for g in goffs[1:]:
                minv = jnp.minimum(minv, idx_v[pl.ds(rbase + g, L)])

            @pl.when(jnp.min(minv) == 0)
            def _fix():
                @pl.loop(0, ngroups)
                def _(gi):
                    g = jnp.minimum(gi * L, last_off)
                    iv = idx_v[pl.ds(rbase + g, L)]

                    @pl.when(jnp.min(iv) == 0)
                    def _group():
                        lane = lax.iota(jnp.int32, L)
                        for l in range(L):
                            zl = jnp.min(jnp.where(lane == l, iv, 1))

                            @pl.when(zl == 0)
                            def _zero_row(l=l):
                                for c in range(D // L):
                                    sl = (g + l, pl.ds(c * L, L))
                                    bufs[b][sl] = jnp.zeros(
                                        (L,), jnp.float32
                                    )

        issue_gathers(0, 0)
        issue_gathers(1, 1)
        issue_gathers(2, 2)

        main = BPT - BPT % NBUF

        @pl.loop(0, main, step=NBUF)
        def _(r0):
            for j in range(NBUF):
                row = r0 + j
                b = j
                wait_gathers(b)
                fixup(b, row)
                issue_write(b, row)
                nxt = row + 3
                bn = (j + 3) % NBUF

                @pl.when(nxt < BPT)
                def _(row=row, nxt=nxt, bn=bn):
                    @pl.when(row >= 1)
                    def _():
                        wait_write(bn)

                    issue_gathers(bn, nxt)

        for row in range(main, BPT):
            b = row % NBUF
            wait_gathers(b)
            fixup(b, row)
            issue_write(b, row)

        for b in range(NBUF):
            wait_write(b)

    return gather_kernel(table_w, idx)[:, :, :D]
